# baseline (speedup 1.0000x reference)
import functools

import jax
import jax.numpy as jnp
from jax import lax
from jax.experimental import pallas as pl
from jax.experimental.pallas import tpu as pltpu
from jax.experimental.pallas import tpu_sc as plsc

N = 10000
E = 320000
D = 128
G = 64

NC = 2
NS = 16
NW = NC * NS
CH = 128
NCH = 80
EPAD = NW * NCH * CH
NROW = EPAD // CH
NPAD = 10240
PADROW = N + 16
RPT = NPAD // NS
DEGW = 16
NBUF = 2

_mesh = plsc.VectorSubcoreMesh(core_axis_name="c", subcore_axis_name="s")


@functools.partial(
    pl.kernel,
    out_type=jax.ShapeDtypeStruct((NC, NPAD, DEGW), jnp.float32),
    mesh=_mesh,
    scratch_types=[
        pltpu.VMEM((NCH, CH), jnp.int32),
        pltpu.VMEM((CH, DEGW), jnp.float32),
        pltpu.VMEM_SHARED((NPAD, DEGW), jnp.float32),
    ],
)
def _sc_degree(dst_hbm, out_hbm, didx, ones_v, acc):
    c = lax.axis_index("c")
    s = lax.axis_index("s")
    w = c * NS + s

    def fill_ones(i, _):
        ones_v[i, pl.ds(0, DEGW)] = jnp.ones((DEGW,), jnp.float32)
        return 0
    lax.fori_loop(0, CH, fill_ones, 0)

    pltpu.sync_copy(dst_hbm.at[pl.ds(w * NCH, NCH)], didx)

    zrow = jnp.zeros((DEGW,), jnp.float32)
    def fill_zeros(i, _):
        ones_v[i, pl.ds(0, DEGW)] = zrow
        return 0
    lax.fori_loop(0, CH, fill_zeros, 0)
    row0 = s * RPT
    def zinit(k, _):
        pltpu.sync_copy(ones_v, acc.at[pl.ds(row0 + k * CH, CH)])
        return 0
    lax.fori_loop(0, RPT // CH, zinit, 0)

    def refill_ones(i, _):
        ones_v[i, pl.ds(0, DEGW)] = jnp.ones((DEGW,), jnp.float32)
        return 0
    lax.fori_loop(0, CH, refill_ones, 0)
    plsc.subcore_barrier()

    def chunk(i, _):
        pltpu.sync_copy(ones_v, acc.at[didx.at[i]], add=True)
        return 0
    lax.fori_loop(0, NCH, chunk, 0)

    plsc.subcore_barrier()
    pltpu.sync_copy(acc.at[pl.ds(row0, RPT)], out_hbm.at[c, pl.ds(row0, RPT)])


@functools.partial(
    pl.kernel,
    out_type=jax.ShapeDtypeStruct((NC, NPAD, D), jnp.float32),
    mesh=_mesh,
    scratch_types=[
        [pltpu.VMEM((CH,), jnp.int32)] * NBUF,
        pltpu.VMEM((CH,), jnp.int32),
        [pltpu.VMEM((CH, D), jnp.float32)] * NBUF,
        pltpu.VMEM_SHARED((NPAD, D), jnp.float32),
        [pltpu.SemaphoreType.DMA] * NBUF,
    ],
)
def _sc_aggregate(h_hbm, src_hbm, dst_hbm, out_hbm,
                  sidx_f, didx_f, rows, acc, sems):
    c = lax.axis_index("c")
    s = lax.axis_index("s")
    w = c * NS + s
    row0 = s * RPT

    def zrows(i, _):
        for j in range(D // 16):
            rows[0][i, pl.ds(j * 16, 16)] = jnp.zeros((16,), jnp.float32)
        return 0
    lax.fori_loop(0, CH, zrows, 0)
    def zinit(k, _):
        pltpu.sync_copy(rows[0], acc.at[pl.ds(row0 + k * CH, CH)])
        return 0
    lax.fori_loop(0, RPT // CH, zinit, 0)
    plsc.subcore_barrier()

    def fire(ch, b):
        pltpu.sync_copy(src_hbm.at[w * NCH + ch], sidx_f[b])
        pltpu.async_copy(h_hbm.at[sidx_f[b]], rows[b], sems[b])

    def drain(ch, b):
        pltpu.make_async_copy(h_hbm.at[sidx_f[b]], rows[b], sems[b]).wait()

    def scat(ch, b):
        pltpu.sync_copy(dst_hbm.at[w * NCH + ch], didx_f)
        pltpu.sync_copy(rows[b], acc.at[didx_f], add=True)

    for b in range(NBUF):
        fire(b, b)

    def step(i, _):
        base = i * NBUF
        for b in range(NBUF):
            ch = base + b
            drain(ch, b)
            scat(ch, b)
            fire(ch + NBUF, b)
        return 0
    lax.fori_loop(0, NCH // NBUF - 2, step, 0)
    base = NCH - 2 * NBUF
    for b in range(NBUF):
        ch = base + b
        drain(ch, b)
        scat(ch, b)
        fire(ch + NBUF, b)
    for b in range(NBUF):
        ch = NCH - NBUF + b
        drain(ch, b)
        scat(ch, b)

    plsc.subcore_barrier()
    pltpu.sync_copy(acc.at[pl.ds(row0, RPT)], out_hbm.at[c, pl.ds(row0, RPT)])


def _mm_body(x_ref, w_ref, o_ref):
    o_ref[...] = jnp.dot(x_ref[...], w_ref[...],
                         preferred_element_type=jnp.float32)


def _tc_matmul(x, w):
    return pl.pallas_call(
        _mm_body,
        out_shape=jax.ShapeDtypeStruct((x.shape[0], w.shape[1]), jnp.float32),
    )(x, w)


def _scale_body(dp_ref, xw_ref, dis_ref, h1p_ref):
    deg = dp_ref[0, 0:N, 0:1] + dp_ref[1, 0:N, 0:1] + 1.0
    dis = lax.rsqrt(deg)
    dis_ref[...] = dis
    h1p_ref[...] = xw_ref[...] * dis


def _tc_scale(deg_partials, xw):
    return pl.pallas_call(
        _scale_body,
        out_shape=[
            jax.ShapeDtypeStruct((N, 1), jnp.float32),
            jax.ShapeDtypeStruct((N, D), jnp.float32),
        ],
    )(deg_partials, xw)


def _layer_body(ap_ref, hp_ref, dis_ref, b_ref, w_ref, o_ref):
    dis = dis_ref[...]
    t = (ap_ref[0, 0:N] + ap_ref[1, 0:N] + hp_ref[...]) * dis + b_ref[...]
    h = jnp.maximum(t, 0.0)
    o_ref[...] = jnp.dot(h, w_ref[...], preferred_element_type=jnp.float32) * dis


def _tc_layer(agg_partials, hp, dis, b2d, w):
    return pl.pallas_call(
        _layer_body,
        out_shape=jax.ShapeDtypeStruct((N, D), jnp.float32),
    )(agg_partials, hp, dis, b2d, w)


def _final_body(ap_ref, hp_ref, dis_ref, b_ref, batch_ref, wl_ref, bl_ref, o_ref):
    h3 = (ap_ref[0, 0:N] + ap_ref[1, 0:N] + hp_ref[...]) * dis_ref[...] + b_ref[...]
    gids = lax.broadcasted_iota(jnp.int32, (N, G), 1)
    onehot = (batch_ref[...] == gids).astype(jnp.float32)
    pooled = lax.dot_general(onehot, h3, (((0,), (0,)), ((), ())),
                             preferred_element_type=jnp.float32)
    cnt = jnp.sum(onehot, axis=0)[:, None]
    g = pooled / jnp.maximum(cnt, 1.0)
    z = jnp.dot(g, wl_ref[...], preferred_element_type=jnp.float32) + bl_ref[...]
    o_ref[...] = 1.0 / (1.0 + jnp.exp(-z))


def _tc_final(agg_partials, hp, dis, b2d, batch2d, wl, bl2d):
    return pl.pallas_call(
        _final_body,
        out_shape=jax.ShapeDtypeStruct((G, 1), jnp.float32),
    )(agg_partials, hp, dis, b2d, batch2d, wl, bl2d)


def kernel(x, edge_index, batch, W1, b1, W2, b2, W3, b3, Wl, bl):
    npad = EPAD - E
    src2d = jnp.concatenate(
        [edge_index[0], jnp.zeros((npad,), edge_index.dtype)]).reshape(NROW, CH)
    dst2d = jnp.concatenate(
        [edge_index[1], jnp.full((npad,), PADROW, edge_index.dtype)]
    ).reshape(NROW, CH)
    batch2d = batch.reshape(N, 1)

    deg_partials = _sc_degree(dst2d)
    xw = _tc_matmul(x, W1)
    dis, hp = _tc_scale(deg_partials, xw)

    agg = _sc_aggregate(hp, src2d, dst2d)
    hp = _tc_layer(agg, hp, dis, b1.reshape(1, D), W2)
    agg = _sc_aggregate(hp, src2d, dst2d)
    hp = _tc_layer(agg, hp, dis, b2.reshape(1, D), W3)
    agg = _sc_aggregate(hp, src2d, dst2d)
    return _tc_final(agg, hp, dis, b3.reshape(1, D), batch2d, Wl,
                     bl.reshape(1, 1))


# speedup vs baseline: 3.0134x; 3.0134x over previous
import functools

import jax
import jax.numpy as jnp
from jax import lax
from jax.experimental import pallas as pl
from jax.experimental.pallas import tpu as pltpu
from jax.experimental.pallas import tpu_sc as plsc

N = 10000
E = 320000
D = 128
G = 64

NC = 2
NS = 16
NW = NC * NS
CH = 128
NCH = 80
ET = E // NW
NCHA = ET // CH
TAIL = ET - NCHA * CH
EPAD = NW * NCH * CH
NROW = EPAD // CH
NPAD = 10240
RPT = NPAD // NS
DEGW = 16
NBUF = 2

_mesh = plsc.VectorSubcoreMesh(core_axis_name="c", subcore_axis_name="s")


@functools.partial(
    pl.kernel,
    out_type=jax.ShapeDtypeStruct((NC, NPAD, DEGW), jnp.float32),
    mesh=_mesh,
    scratch_types=[
        pltpu.VMEM((NCH, CH), jnp.int32),
        pltpu.VMEM((CH, DEGW), jnp.float32),
        pltpu.VMEM_SHARED((NPAD, DEGW), jnp.float32),
    ],
)
def _sc_degree(dst_hbm, out_hbm, didx, ones_v, acc):
    c = lax.axis_index("c")
    s = lax.axis_index("s")
    w = c * NS + s

    def fill_ones(i, _):
        ones_v[i, pl.ds(0, DEGW)] = jnp.ones((DEGW,), jnp.float32)
        return 0
    lax.fori_loop(0, CH, fill_ones, 0)

    pltpu.sync_copy(dst_hbm.at[pl.ds(w * NCH, NCH)], didx)

    zrow = jnp.zeros((DEGW,), jnp.float32)
    def fill_zeros(i, _):
        ones_v[i, pl.ds(0, DEGW)] = zrow
        return 0
    lax.fori_loop(0, CH, fill_zeros, 0)
    row0 = s * RPT
    def zinit(k, _):
        pltpu.sync_copy(ones_v, acc.at[pl.ds(row0 + k * CH, CH)])
        return 0
    lax.fori_loop(0, RPT // CH, zinit, 0)

    def refill_ones(i, _):
        ones_v[i, pl.ds(0, DEGW)] = jnp.ones((DEGW,), jnp.float32)
        return 0
    lax.fori_loop(0, CH, refill_ones, 0)
    plsc.subcore_barrier()

    def chunk(i, _):
        pltpu.sync_copy(ones_v, acc.at[didx.at[i]], add=True)
        return 0
    lax.fori_loop(0, NCH, chunk, 0)

    plsc.subcore_barrier()
    pltpu.sync_copy(acc.at[pl.ds(row0, RPT)], out_hbm.at[c, pl.ds(row0, RPT)])


@functools.partial(
    pl.kernel,
    out_type=jax.ShapeDtypeStruct((NC, NPAD, D), jnp.float32),
    mesh=_mesh,
    scratch_types=[
        [pltpu.VMEM((CH,), jnp.int32)] * NBUF,
        pltpu.VMEM((CH,), jnp.int32),
        pltpu.VMEM((TAIL,), jnp.int32),
        [pltpu.VMEM((CH, D), jnp.float32)] * NBUF,
        pltpu.VMEM((TAIL, D), jnp.float32),
        pltpu.VMEM_SHARED((NPAD, D), jnp.float32),
        [pltpu.SemaphoreType.DMA] * NBUF,
        pltpu.SemaphoreType.DMA,
    ],
)
def _sc_aggregate(h_hbm, src_hbm, dst_hbm, out_hbm,
                  sidx_f, didx_f, tidx, rows, rows_t, acc, sems, tsem):
    c = lax.axis_index("c")
    s = lax.axis_index("s")
    w = c * NS + s
    row0 = s * RPT
    base = w * ET

    def zrows(i, _):
        for j in range(D // 16):
            rows[0][i, pl.ds(j * 16, 16)] = jnp.zeros((16,), jnp.float32)
        return 0
    lax.fori_loop(0, CH, zrows, 0)
    def zinit(k, _):
        pltpu.sync_copy(rows[0], acc.at[pl.ds(row0 + k * CH, CH)])
        return 0
    lax.fori_loop(0, RPT // CH, zinit, 0)
    plsc.subcore_barrier()

    def fire(ch, b):
        pltpu.sync_copy(src_hbm.at[pl.ds(base + ch * CH, CH)], sidx_f[b])
        pltpu.async_copy(h_hbm.at[sidx_f[b]], rows[b], sems[b])

    def drain(ch, b):
        pltpu.make_async_copy(h_hbm.at[sidx_f[b]], rows[b], sems[b]).wait()

    def scat(ch, b):
        pltpu.sync_copy(dst_hbm.at[pl.ds(base + ch * CH, CH)], didx_f)
        pltpu.sync_copy(rows[b], acc.at[didx_f], add=True)

    for b in range(NBUF):
        fire(b, b)

    def step(i, _):
        bs = i * NBUF
        for b in range(NBUF):
            ch = bs + b
            drain(ch, b)
            scat(ch, b)
            fire(ch + NBUF, b)
        return 0
    lax.fori_loop(0, NCHA // NBUF - 2, step, 0)
    bs = NCHA - 2 * NBUF
    for b in range(NBUF):
        ch = bs + b
        drain(ch, b)
        scat(ch, b)
        fire(ch + NBUF, b)
    for b in range(NBUF):
        ch = NCHA - NBUF + b
        drain(ch, b)
        scat(ch, b)

    toff = base + NCHA * CH
    pltpu.sync_copy(src_hbm.at[pl.ds(toff, TAIL)], tidx)
    pltpu.async_copy(h_hbm.at[tidx], rows_t, tsem).wait()
    pltpu.sync_copy(dst_hbm.at[pl.ds(toff, TAIL)], tidx)
    pltpu.sync_copy(rows_t, acc.at[tidx], add=True)

    plsc.subcore_barrier()
    pltpu.sync_copy(acc.at[pl.ds(row0, RPT)], out_hbm.at[c, pl.ds(row0, RPT)])


def _mm_body(x_ref, w_ref, o_ref):
    o_ref[...] = jnp.dot(x_ref[...], w_ref[...],
                         preferred_element_type=jnp.float32)


def _tc_matmul(x, w):
    return pl.pallas_call(
        _mm_body,
        out_shape=jax.ShapeDtypeStruct((x.shape[0], w.shape[1]), jnp.float32),
    )(x, w)


def _scale_body(dp_ref, xw_ref, dis_ref, h1p_ref):
    deg = dp_ref[0, 0:N, 0:1] + dp_ref[1, 0:N, 0:1] + 1.0
    dis = lax.rsqrt(deg)
    dis_ref[...] = dis
    h1p_ref[...] = xw_ref[...] * dis


def _tc_scale(deg_partials, xw):
    return pl.pallas_call(
        _scale_body,
        out_shape=[
            jax.ShapeDtypeStruct((N, 1), jnp.float32),
            jax.ShapeDtypeStruct((N, D), jnp.float32),
        ],
    )(deg_partials, xw)


def _layer_body(ap_ref, hp_ref, dis_ref, b_ref, w_ref, o_ref):
    dis = dis_ref[...]
    t = (ap_ref[0, 0:N] + ap_ref[1, 0:N] + hp_ref[...]) * dis + b_ref[...]
    h = jnp.maximum(t, 0.0)
    o_ref[...] = jnp.dot(h, w_ref[...], preferred_element_type=jnp.float32) * dis


def _tc_layer(agg_partials, hp, dis, b2d, w):
    return pl.pallas_call(
        _layer_body,
        out_shape=jax.ShapeDtypeStruct((N, D), jnp.float32),
    )(agg_partials, hp, dis, b2d, w)


def _final_body(ap_ref, hp_ref, dis_ref, b_ref, batch_ref, wl_ref, bl_ref, o_ref):
    h3 = (ap_ref[0, 0:N] + ap_ref[1, 0:N] + hp_ref[...]) * dis_ref[...] + b_ref[...]
    gids = lax.broadcasted_iota(jnp.int32, (N, G), 1)
    onehot = (batch_ref[...] == gids).astype(jnp.float32)
    pooled = lax.dot_general(onehot, h3, (((0,), (0,)), ((), ())),
                             preferred_element_type=jnp.float32)
    cnt = jnp.sum(onehot, axis=0)[:, None]
    g = pooled / jnp.maximum(cnt, 1.0)
    z = jnp.dot(g, wl_ref[...], preferred_element_type=jnp.float32) + bl_ref[...]
    o_ref[...] = 1.0 / (1.0 + jnp.exp(-z))


def _tc_final(agg_partials, hp, dis, b2d, batch2d, wl, bl2d):
    return pl.pallas_call(
        _final_body,
        out_shape=jax.ShapeDtypeStruct((G, 1), jnp.float32),
    )(agg_partials, hp, dis, b2d, batch2d, wl, bl2d)


def kernel(x, edge_index, batch, W1, b1, W2, b2, W3, b3, Wl, bl):
    npad = EPAD - E
    dst2d = jnp.concatenate(
        [edge_index[1], jnp.full((npad,), N + 16, edge_index.dtype)]).reshape(NROW, CH)
    src1d = edge_index[0]
    dst1d = edge_index[1]
    batch2d = batch.reshape(N, 1)

    deg_partials = _sc_degree(dst2d)
    xw = _tc_matmul(x, W1)
    dis, hp = _tc_scale(deg_partials, xw)

    agg = _sc_aggregate(hp, src1d, dst1d)
    hp = _tc_layer(agg, hp, dis, b1.reshape(1, D), W2)
    agg = _sc_aggregate(hp, src1d, dst1d)
    hp = _tc_layer(agg, hp, dis, b2.reshape(1, D), W3)
    agg = _sc_aggregate(hp, src1d, dst1d)
    return _tc_final(agg, hp, dis, b3.reshape(1, D), batch2d, Wl,
                     bl.reshape(1, 1))


# speedup vs baseline: 3.8550x; 1.2793x over previous
import functools

import jax
import jax.numpy as jnp
from jax import lax
from jax.experimental import pallas as pl
from jax.experimental.pallas import tpu as pltpu
from jax.experimental.pallas import tpu_sc as plsc

N = 10000
E = 320000
D = 128
G = 64

NC = 2
NS = 16
NW = NC * NS
CH = 128
NCH = 80
ET = E // NW
NCHA = ET // CH
TAIL = ET - NCHA * CH
EPAD = NW * NCH * CH
NROW = EPAD // CH
NPAD = 10240
RPT = NPAD // NS
DEGW = 16
NBUF = 2

_mesh = plsc.VectorSubcoreMesh(core_axis_name="c", subcore_axis_name="s")


@functools.partial(
    pl.kernel,
    out_type=jax.ShapeDtypeStruct((NC, NPAD, DEGW), jnp.float32),
    mesh=_mesh,
    scratch_types=[
        pltpu.VMEM((NCH, CH), jnp.int32),
        pltpu.VMEM((CH, DEGW), jnp.float32),
        pltpu.VMEM_SHARED((NPAD, DEGW), jnp.float32),
    ],
)
def _sc_degree(dst_hbm, out_hbm, didx, ones_v, acc):
    c = lax.axis_index("c")
    s = lax.axis_index("s")
    w = c * NS + s

    def fill_ones(i, _):
        ones_v[i, pl.ds(0, DEGW)] = jnp.ones((DEGW,), jnp.float32)
        return 0
    lax.fori_loop(0, CH, fill_ones, 0)

    pltpu.sync_copy(dst_hbm.at[pl.ds(w * NCH, NCH)], didx)

    zrow = jnp.zeros((DEGW,), jnp.float32)
    def fill_zeros(i, _):
        ones_v[i, pl.ds(0, DEGW)] = zrow
        return 0
    lax.fori_loop(0, CH, fill_zeros, 0)
    row0 = s * RPT
    def zinit(k, _):
        pltpu.sync_copy(ones_v, acc.at[pl.ds(row0 + k * CH, CH)])
        return 0
    lax.fori_loop(0, RPT // CH, zinit, 0)

    def refill_ones(i, _):
        ones_v[i, pl.ds(0, DEGW)] = jnp.ones((DEGW,), jnp.float32)
        return 0
    lax.fori_loop(0, CH, refill_ones, 0)
    plsc.subcore_barrier()

    def chunk(i, _):
        pltpu.sync_copy(ones_v, acc.at[didx.at[i]], add=True)
        return 0
    lax.fori_loop(0, NCH, chunk, 0)

    plsc.subcore_barrier()
    pltpu.sync_copy(acc.at[pl.ds(row0, RPT)], out_hbm.at[c, pl.ds(row0, RPT)])


@functools.partial(
    pl.kernel,
    out_type=jax.ShapeDtypeStruct((NC, NPAD, D), jnp.float32),
    mesh=_mesh,
    scratch_types=[
        [pltpu.VMEM((CH,), jnp.int32)] * (2 * NBUF),
        [pltpu.VMEM((CH,), jnp.int32)] * (2 * NBUF),
        pltpu.VMEM((TAIL,), jnp.int32),
        [pltpu.VMEM((CH, D), jnp.float32)] * NBUF,
        pltpu.VMEM((TAIL, D), jnp.float32),
        pltpu.VMEM_SHARED((NPAD, D), jnp.float32),
        [pltpu.SemaphoreType.DMA] * (2 * NBUF),
        [pltpu.SemaphoreType.DMA] * NBUF,
        pltpu.SemaphoreType.DMA,
    ],
)
def _sc_aggregate(h_hbm, src_hbm, dst_hbm, out_hbm,
                  sidx, didx, tidx, rows, rows_t, acc, isems, gsems, tsem):
    c = lax.axis_index("c")
    s = lax.axis_index("s")
    w = c * NS + s
    row0 = s * RPT
    base = w * ET
    NI = 2 * NBUF

    def zrows(i, _):
        for j in range(D // 16):
            rows[0][i, pl.ds(j * 16, 16)] = jnp.zeros((16,), jnp.float32)
        return 0
    lax.fori_loop(0, CH, zrows, 0)
    def zinit(k, _):
        pltpu.sync_copy(rows[0], acc.at[pl.ds(row0 + k * CH, CH)])
        return 0
    lax.fori_loop(0, RPT // CH, zinit, 0)
    plsc.subcore_barrier()

    def iload(ch, ib):
        pltpu.async_copy(src_hbm.at[pl.ds(base + ch * CH, CH)], sidx[ib],
                         isems[ib])
        pltpu.async_copy(dst_hbm.at[pl.ds(base + ch * CH, CH)], didx[ib],
                         isems[ib])

    def idrain(ch, ib):
        pltpu.make_async_copy(src_hbm.at[pl.ds(base + ch * CH, CH)], sidx[ib],
                              isems[ib]).wait()
        pltpu.make_async_copy(dst_hbm.at[pl.ds(base + ch * CH, CH)], didx[ib],
                              isems[ib]).wait()

    def gfire(ch, ib, b):
        pltpu.async_copy(h_hbm.at[sidx[ib]], rows[b], gsems[b])

    def gdrain(ch, ib, b):
        pltpu.make_async_copy(h_hbm.at[sidx[ib]], rows[b],
                              gsems[b]).wait()

    def scat(ib, b):
        pltpu.sync_copy(rows[b], acc.at[didx[ib]], add=True)

    def visit(ch, j, fire_iload, fire_gather):
        b = j % NBUF
        ib = j % NI
        ib2 = (j + NBUF) % NI
        gdrain(ch, ib, b)
        scat(ib, b)
        if fire_iload:
            iload(ch + NI, ib)
        if fire_gather:
            idrain(ch + NBUF, ib2)
            gfire(ch + NBUF, ib2, b)

    for ib in range(NI):
        iload(ib, ib)
    for ch in range(NBUF):
        idrain(ch, ch)
        gfire(ch, ch, ch)

    def step(i, _):
        bs = i * NI
        for j in range(NI):
            visit(bs + j, j, True, True)
        return 0
    nmain = (NCHA - NI) // NI
    lax.fori_loop(0, nmain, step, 0)
    for ch in range(nmain * NI, NCHA):
        visit(ch, ch % NI, ch + NI < NCHA, ch + NBUF < NCHA)

    toff = base + NCHA * CH
    pltpu.sync_copy(src_hbm.at[pl.ds(toff, TAIL)], tidx)
    pltpu.async_copy(h_hbm.at[tidx], rows_t, tsem).wait()
    pltpu.sync_copy(dst_hbm.at[pl.ds(toff, TAIL)], tidx)
    pltpu.sync_copy(rows_t, acc.at[tidx], add=True)

    plsc.subcore_barrier()
    pltpu.sync_copy(acc.at[pl.ds(row0, RPT)], out_hbm.at[c, pl.ds(row0, RPT)])


def _mm_body(x_ref, w_ref, o_ref):
    o_ref[...] = jnp.dot(x_ref[...], w_ref[...],
                         preferred_element_type=jnp.float32)


def _tc_matmul(x, w):
    return pl.pallas_call(
        _mm_body,
        out_shape=jax.ShapeDtypeStruct((x.shape[0], w.shape[1]), jnp.float32),
    )(x, w)


def _scale_body(dp_ref, xw_ref, dis_ref, h1p_ref):
    deg = dp_ref[0, 0:N, 0:1] + dp_ref[1, 0:N, 0:1] + 1.0
    dis = lax.rsqrt(deg)
    dis_ref[...] = dis
    h1p_ref[...] = xw_ref[...] * dis


def _tc_scale(deg_partials, xw):
    return pl.pallas_call(
        _scale_body,
        out_shape=[
            jax.ShapeDtypeStruct((N, 1), jnp.float32),
            jax.ShapeDtypeStruct((N, D), jnp.float32),
        ],
    )(deg_partials, xw)


def _layer_body(ap_ref, hp_ref, dis_ref, b_ref, w_ref, o_ref):
    dis = dis_ref[...]
    t = (ap_ref[0, 0:N] + ap_ref[1, 0:N] + hp_ref[...]) * dis + b_ref[...]
    h = jnp.maximum(t, 0.0)
    o_ref[...] = jnp.dot(h, w_ref[...], preferred_element_type=jnp.float32) * dis


def _tc_layer(agg_partials, hp, dis, b2d, w):
    return pl.pallas_call(
        _layer_body,
        out_shape=jax.ShapeDtypeStruct((N, D), jnp.float32),
    )(agg_partials, hp, dis, b2d, w)


def _final_body(ap_ref, hp_ref, dis_ref, b_ref, batch_ref, wl_ref, bl_ref, o_ref):
    h3 = (ap_ref[0, 0:N] + ap_ref[1, 0:N] + hp_ref[...]) * dis_ref[...] + b_ref[...]
    gids = lax.broadcasted_iota(jnp.int32, (N, G), 1)
    onehot = (batch_ref[...] == gids).astype(jnp.float32)
    pooled = lax.dot_general(onehot, h3, (((0,), (0,)), ((), ())),
                             preferred_element_type=jnp.float32)
    cnt = jnp.sum(onehot, axis=0)[:, None]
    g = pooled / jnp.maximum(cnt, 1.0)
    z = jnp.dot(g, wl_ref[...], preferred_element_type=jnp.float32) + bl_ref[...]
    o_ref[...] = 1.0 / (1.0 + jnp.exp(-z))


def _tc_final(agg_partials, hp, dis, b2d, batch2d, wl, bl2d):
    return pl.pallas_call(
        _final_body,
        out_shape=jax.ShapeDtypeStruct((G, 1), jnp.float32),
    )(agg_partials, hp, dis, b2d, batch2d, wl, bl2d)


def kernel(x, edge_index, batch, W1, b1, W2, b2, W3, b3, Wl, bl):
    npad = EPAD - E
    dst2d = jnp.concatenate(
        [edge_index[1], jnp.full((npad,), N + 16, edge_index.dtype)]).reshape(NROW, CH)
    src1d = edge_index[0]
    dst1d = edge_index[1]
    batch2d = batch.reshape(N, 1)

    deg_partials = _sc_degree(dst2d)
    xw = _tc_matmul(x, W1)
    dis, hp = _tc_scale(deg_partials, xw)

    agg = _sc_aggregate(hp, src1d, dst1d)
    hp = _tc_layer(agg, hp, dis, b1.reshape(1, D), W2)
    agg = _sc_aggregate(hp, src1d, dst1d)
    hp = _tc_layer(agg, hp, dis, b2.reshape(1, D), W3)
    agg = _sc_aggregate(hp, src1d, dst1d)
    return _tc_final(agg, hp, dis, b3.reshape(1, D), batch2d, Wl,
                     bl.reshape(1, 1))
